# baseline (speedup 1.0000x reference)
import functools

import jax
import jax.numpy as jnp
from jax import lax
from jax.experimental import pallas as pl
from jax.experimental.pallas import tpu as pltpu
from jax.experimental.pallas import tpu_sc as plsc

B, L, D = 4096, 200, 64
NW = 32
BW = B // NW

_MESH = plsc.VectorSubcoreMesh(core_axis_name="c", subcore_axis_name="s")


@functools.partial(
    pl.kernel,
    mesh=_MESH,
    out_type=jax.ShapeDtypeStruct((L, 8, NW, 8, 128), jnp.float32),
    scratch_types=[
        pltpu.VMEM((L, BW), jnp.int32),
        pltpu.VMEM((BW, D), jnp.float32),
        pltpu.VMEM((BW, D), jnp.float32),
        pltpu.VMEM((8, 8, 128), jnp.float32),
        pltpu.VMEM((8, 8, 128), jnp.float32),
        pltpu.SemaphoreType.DMA,
        pltpu.SemaphoreType.DMA,
        pltpu.SemaphoreType.DMA,
        pltpu.SemaphoreType.DMA,
    ],
    compiler_params=pltpu.CompilerParams(use_tc_tiling_on_sc=False,
                                         needs_layout_passes=False),
)
def _emb_gather(xt_hbm, table_hbm, out_hbm,
                idx_t, rows0, rows1, t0, t1, g0, g1, o0, o1):
    wid = lax.axis_index("s") * 2 + lax.axis_index("c")
    pltpu.sync_copy(xt_hbm.at[:, pl.ds(wid * BW, BW)], idx_t)

    def gather(l, rows, g):
        pltpu.async_copy(table_hbm.at[idx_t.at[l]], rows, g)

    def wait_gather(rows, g):
        pltpu.make_async_copy(table_hbm.at[idx_t.at[0]], rows, g).wait()

    def store(l, t, o):
        pltpu.async_copy(t, out_hbm.at[l, :, wid], o)

    def wait_store(t, o):
        pltpu.make_async_copy(t, out_hbm.at[0, :, wid], o).wait()

    lanes = lax.iota(jnp.int32, 16)

    def turn(rows, t):
        def tbody(d, _):
            dh = d // 8
            dl = d % 8
            col = jnp.full((16,), d, jnp.int32)
            for j in range(8):
                v = plsc.load_gather(rows, [lanes + 16 * j, col])
                t[dh, dl, pl.ds(16 * j, 16)] = v
            return 0

        lax.fori_loop(0, D, tbody, 0)

    gather(0, rows0, g0)
    wait_gather(rows0, g0)
    gather(1, rows1, g1)
    turn(rows0, t0)
    store(0, t0, o0)
    wait_gather(rows1, g1)
    gather(2, rows0, g0)
    turn(rows1, t1)
    store(1, t1, o1)

    def body(k, _):
        l0 = 2 * k
        wait_gather(rows0, g0)
        gather(l0 + 1, rows1, g1)
        wait_store(t0, o0)
        turn(rows0, t0)
        store(l0, t0, o0)
        wait_gather(rows1, g1)

        @pl.when(l0 + 2 < L)
        def _():
            gather(l0 + 2, rows0, g0)

        wait_store(t1, o1)
        turn(rows1, t1)
        store(l0 + 1, t1, o1)
        return 0

    lax.fori_loop(1, L // 2, body, 0)
    wait_store(t0, o0)
    wait_store(t1, o1)


def kernel(x, table):
    xt = x.T.astype(jnp.int32)
    out5 = _emb_gather(xt, table)
    return out5.transpose(2, 4, 0, 1, 3).reshape(B, L, D)


# speedup vs baseline: 4.3997x; 4.3997x over previous
import functools

import jax
import jax.numpy as jnp
from jax import lax
from jax.experimental import pallas as pl
from jax.experimental.pallas import tpu as pltpu
from jax.experimental.pallas import tpu_sc as plsc

B, L, D = 4096, 200, 64
NW = 32
BW = B // NW

_MESH = plsc.VectorSubcoreMesh(core_axis_name="c", subcore_axis_name="s")


@functools.partial(
    pl.kernel,
    mesh=_MESH,
    out_type=jax.ShapeDtypeStruct((L, 8, NW, 8, 128), jnp.float32),
    scratch_types=[
        pltpu.VMEM((L, BW), jnp.int32),
        pltpu.VMEM((BW, D), jnp.float32),
        pltpu.VMEM((BW, D), jnp.float32),
        pltpu.VMEM((8, 8, 128), jnp.float32),
        pltpu.VMEM((8, 8, 128), jnp.float32),
        pltpu.SemaphoreType.DMA,
        pltpu.SemaphoreType.DMA,
        pltpu.SemaphoreType.DMA,
        pltpu.SemaphoreType.DMA,
    ],
    compiler_params=pltpu.CompilerParams(use_tc_tiling_on_sc=False,
                                         needs_layout_passes=False),
)
def _emb_gather(xt_hbm, table_hbm, out_hbm,
                idx_t, rows0, rows1, t0, t1, g0, g1, o0, o1):
    wid = lax.axis_index("s") * 2 + lax.axis_index("c")
    pltpu.sync_copy(xt_hbm.at[:, pl.ds(wid * BW, BW)], idx_t)

    def gather(l, rows, g):
        pltpu.async_copy(table_hbm.at[idx_t.at[l]], rows, g)

    def wait_gather(rows, g):
        pltpu.make_async_copy(table_hbm.at[idx_t.at[0]], rows, g).wait()

    def store(l, t, o):
        pltpu.async_copy(t, out_hbm.at[l, :, wid], o)

    def wait_store(t, o):
        pltpu.make_async_copy(t, out_hbm.at[0, :, wid], o).wait()

    lanes_j = [lax.iota(jnp.int32, 16) + 16 * j for j in range(8)]

    def turn(rows, t):
        @functools.partial(plsc.parallel_loop, 0, D, unroll=2)
        def _tbody(d):
            dh = d // 8
            dl = d % 8
            col = jnp.full((16,), d, jnp.int32)
            for j in range(8):
                v = plsc.load_gather(rows, [lanes_j[j], col])
                t[dh, dl, pl.ds(16 * j, 16)] = v

    gather(0, rows0, g0)
    wait_gather(rows0, g0)
    gather(1, rows1, g1)
    turn(rows0, t0)
    store(0, t0, o0)
    wait_gather(rows1, g1)
    gather(2, rows0, g0)
    turn(rows1, t1)
    store(1, t1, o1)

    def body(k, _):
        l0 = 2 * k
        wait_gather(rows0, g0)
        gather(l0 + 1, rows1, g1)
        wait_store(t0, o0)
        turn(rows0, t0)
        store(l0, t0, o0)
        wait_gather(rows1, g1)

        @pl.when(l0 + 2 < L)
        def _():
            gather(l0 + 2, rows0, g0)

        wait_store(t1, o1)
        turn(rows1, t1)
        store(l0 + 1, t1, o1)
        return 0

    lax.fori_loop(1, L // 2, body, 0)
    wait_store(t0, o0)
    wait_store(t1, o1)


def kernel(x, table):
    xt = x.T.astype(jnp.int32)
    out5 = _emb_gather(xt, table)
    return out5.transpose(2, 4, 0, 1, 3).reshape(B, L, D)
